# baseline (speedup 1.0000x reference)
import jax
import jax.numpy as jnp
from jax import lax
from jax.experimental import pallas as pl
from jax.experimental.pallas import tpu as pltpu
from jax.experimental.pallas import tpu_sc as plsc

VOCAB = 100000
MAXLEN = 60
D = 144
BATCH = 16384
SEQ = 50
NV = D // 16

NC, NS = 2, 16
NW = NC * NS
B_PER_W = BATCH // NW
H = 128
NH = B_PER_W // H
NSTEP = SEQ * NH


def _hsum(v, lanes):
    for s in (8, 4, 2, 1):
        v = v + v.at[lanes ^ s].get(mode="promise_in_bounds")
    return v


def _vrsqrt(x):
    i = lax.bitcast_convert_type(x, jnp.int32)
    y = lax.bitcast_convert_type(jnp.int32(0x5F3759DF) - (i >> 1), jnp.float32)
    for _ in range(2):
        y = y * (1.5 - 0.5 * x * y * y)
    return y


def _tree_sum(vs):
    while len(vs) > 1:
        vs = [vs[i] + vs[i + 1] for i in range(0, len(vs) - 1, 2)] + (
            [vs[-1]] if len(vs) % 2 else [])
    return vs[0]


def _sc_body(xt_hbm, tok_hbm, pos_hbm, gamma_hbm, beta_hbm, out_hbm,
             idx_v, in0_v, in1_v, out0_v, out1_v, pos_v, gb_v,
             gsem0, gsem1, ssem):
    wid = lax.axis_index("s") * NC + lax.axis_index("c")
    bbase = wid * B_PER_W
    in_bufs = (in0_v, in1_v)
    out_bufs = (out0_v, out1_v)
    gsems = (gsem0, gsem1)

    pltpu.sync_copy(xt_hbm.at[:, pl.ds(bbase, B_PER_W)], idx_v)
    pltpu.sync_copy(pos_hbm, pos_v)
    pltpu.sync_copy(gamma_hbm, gb_v.at[0])
    pltpu.sync_copy(beta_hbm, gb_v.at[1])

    g_vecs = [gb_v[0, pl.ds(k * 16, 16)] for k in range(NV)]
    b_vecs = [gb_v[1, pl.ds(k * 16, 16)] for k in range(NV)]
    lanes = lax.iota(jnp.int32, 16)

    def gather(g, b):
        p = g >> 2
        h = g & 3
        pltpu.async_copy(
            tok_hbm.at[idx_v.at[p, pl.ds(h * H, H)]], in_bufs[b], gsems[b])

    def gather_wait(b):
        pltpu.make_async_copy(
            tok_hbm.at[idx_v.at[0, pl.ds(0, H)]], in_bufs[b], gsems[b]).wait()

    def store_wait():
        pltpu.make_async_copy(
            out_hbm.at[0, :, pl.ds(0, H)], out0_v, ssem).wait()

    gather(0, 0)
    gather(1, 1)

    @pl.loop(0, NSTEP, step=2)
    def step_loop(g0):
        for b in range(2):
            g = g0 + b
            p = g >> 2
            h = g & 3
            gather_wait(b)

            @pl.when(g >= 2)
            def _():
                store_wait()

            in_v = in_bufs[b]
            out_v = out_bufs[b]
            pvs = [pos_v[p, pl.ds(k * 16, 16)] for k in range(NV)]

            @pl.loop(0, H)
            def row_loop(t):
                vs = [in_v[t, pl.ds(k * 16, 16)] + pvs[k] for k in range(NV)]
                mean = _hsum(_tree_sum(vs), lanes) * (1.0 / D)
                q = _hsum(_tree_sum([v * v for v in vs]), lanes) * (1.0 / D)
                r = _vrsqrt(q - mean * mean + 1e-5)
                tv = jnp.broadcast_to(t, (16,))
                for k in range(NV):
                    o = (vs[k] - mean) * (r * g_vecs[k]) + b_vecs[k]
                    plsc.store_scatter(out_v, [lanes + (16 * k), tv], o)

            @pl.when(g + 2 < NSTEP)
            def _():
                gather(g + 2, b)

            pltpu.async_copy(
                out_v, out_hbm.at[p, :, pl.ds(bbase + h * H, H)], ssem)

    store_wait()
    store_wait()


@jax.jit
def kernel(x, tok_table, pos_table, gamma, beta):
    x_t = x.T.astype(jnp.int32)
    mesh = plsc.VectorSubcoreMesh(core_axis_name="c", subcore_axis_name="s")
    out_t = pl.kernel(
        _sc_body,
        out_type=jax.ShapeDtypeStruct((SEQ, D, BATCH), jnp.float32),
        mesh=mesh,
        compiler_params=pltpu.CompilerParams(
            use_tc_tiling_on_sc=False, needs_layout_passes=False),
        scratch_types=[
            pltpu.VMEM((SEQ, B_PER_W), jnp.int32),
            pltpu.VMEM((H, D), jnp.float32),
            pltpu.VMEM((H, D), jnp.float32),
            pltpu.VMEM((D, H), jnp.float32),
            pltpu.VMEM((D, H), jnp.float32),
            pltpu.VMEM((MAXLEN, D), jnp.float32),
            pltpu.VMEM((2, D), jnp.float32),
            pltpu.SemaphoreType.DMA,
            pltpu.SemaphoreType.DMA,
            pltpu.SemaphoreType.DMA,
        ],
    )(x_t, tok_table, pos_table, gamma, beta)
    return out_t.transpose(2, 0, 1)


# speedup vs baseline: 1.5587x; 1.5587x over previous
import jax
import jax.numpy as jnp
from jax import lax
from jax.experimental import pallas as pl
from jax.experimental.pallas import tpu as pltpu
from jax.experimental.pallas import tpu_sc as plsc

VOCAB = 100000
MAXLEN = 60
D = 144
BATCH = 16384
SEQ = 50
NV = D // 16

NC, NS = 2, 16
NW = NC * NS
B_PER_W = BATCH // NW
H = 128
NH = B_PER_W // H
NSTEP = SEQ * NH


def _hsum(v, lanes):
    for s in (8, 4, 2, 1):
        v = v + v.at[lanes ^ s].get(mode="promise_in_bounds")
    return v


def _vrsqrt(x):
    i = lax.bitcast_convert_type(x, jnp.int32)
    y = lax.bitcast_convert_type(jnp.int32(0x5F3759DF) - (i >> 1), jnp.float32)
    for _ in range(2):
        y = y * (1.5 - 0.5 * x * y * y)
    return y


def _tree_sum(vs):
    while len(vs) > 1:
        vs = [vs[i] + vs[i + 1] for i in range(0, len(vs) - 1, 2)] + (
            [vs[-1]] if len(vs) % 2 else [])
    return vs[0]


def _sc_body(xt_hbm, tok_hbm, pos_hbm, gamma_hbm, beta_hbm, out_hbm,
             idx_v, in0_v, in1_v, out0_v, out1_v, pos_v, gb_v,
             gsem0, gsem1, ssem):
    wid = lax.axis_index("s") * NC + lax.axis_index("c")
    bbase = wid * B_PER_W
    in_bufs = (in0_v, in1_v)
    out_bufs = (out0_v, out1_v)
    gsems = (gsem0, gsem1)

    pltpu.sync_copy(xt_hbm.at[:, pl.ds(bbase, B_PER_W)], idx_v)
    pltpu.sync_copy(pos_hbm, pos_v)
    pltpu.sync_copy(gamma_hbm, gb_v.at[0])
    pltpu.sync_copy(beta_hbm, gb_v.at[1])

    g_vecs = [gb_v[0, pl.ds(k * 16, 16)] for k in range(NV)]
    b_vecs = [gb_v[1, pl.ds(k * 16, 16)] for k in range(NV)]
    lanes = lax.iota(jnp.int32, 16)

    def gather(g, b):
        p = g >> 2
        h = g & 3
        pltpu.async_copy(
            tok_hbm.at[idx_v.at[p, pl.ds(h * H, H)]], in_bufs[b], gsems[b])

    def gather_wait(b):
        pltpu.make_async_copy(
            tok_hbm.at[idx_v.at[0, pl.ds(0, H)]], in_bufs[b], gsems[b]).wait()

    def store_wait():
        pltpu.make_async_copy(
            out_hbm.at[0, :, pl.ds(0, H)],
            out0_v.at[:, pl.ds(0, H)], ssem).wait()

    gather(0, 0)
    gather(1, 1)

    @pl.loop(0, NSTEP, step=2)
    def step_loop(g0):
        for b in range(2):
            g = g0 + b
            p = g >> 2
            h = g & 3
            gather_wait(b)

            @pl.when(g >= 2)
            def _():
                store_wait()

            in_v = in_bufs[b]
            out_v = out_bufs[b]
            pvs = [pos_v[p, pl.ds(k * 16, 16)] for k in range(NV)]

            @pl.loop(0, H, unroll=2)
            def row_loop(t):
                vs = [in_v[t, pl.ds(k * 16, 16)] + pvs[k] for k in range(NV)]
                mean = _hsum(_tree_sum(vs), lanes) * (1.0 / D)
                q = _hsum(_tree_sum([v * v for v in vs]), lanes) * (1.0 / D)
                r = _vrsqrt(q - mean * mean + 1e-5)
                tv = jnp.broadcast_to(t, (16,))
                for k in range(NV):
                    o = (vs[k] - mean) * (r * g_vecs[k]) + b_vecs[k]
                    plsc.store_scatter(out_v, [lanes + (16 * k), tv], o)

            @pl.when(g + 2 < NSTEP)
            def _():
                gather(g + 2, b)

            pltpu.async_copy(
                out_v.at[:, pl.ds(0, H)],
                out_hbm.at[p, :, pl.ds(bbase + h * H, H)], ssem)

    store_wait()
    store_wait()


@jax.jit
def kernel(x, tok_table, pos_table, gamma, beta):
    x_t = x.T.astype(jnp.int32)
    mesh = plsc.VectorSubcoreMesh(core_axis_name="c", subcore_axis_name="s")
    out_t = pl.kernel(
        _sc_body,
        out_type=jax.ShapeDtypeStruct((SEQ, D, BATCH), jnp.float32),
        mesh=mesh,
        compiler_params=pltpu.CompilerParams(
            use_tc_tiling_on_sc=False, needs_layout_passes=False),
        scratch_types=[
            pltpu.VMEM((SEQ, B_PER_W), jnp.int32),
            pltpu.VMEM((H, D), jnp.float32),
            pltpu.VMEM((H, D), jnp.float32),
            pltpu.VMEM((D, H + 1), jnp.float32),
            pltpu.VMEM((D, H + 1), jnp.float32),
            pltpu.VMEM((MAXLEN, D), jnp.float32),
            pltpu.VMEM((2, D), jnp.float32),
            pltpu.SemaphoreType.DMA,
            pltpu.SemaphoreType.DMA,
            pltpu.SemaphoreType.DMA,
        ],
    )(x_t, tok_table, pos_table, gamma, beta)
    return out_t.transpose(2, 0, 1)


# speedup vs baseline: 1.9028x; 1.2207x over previous
import jax
import jax.numpy as jnp
from jax import lax
from jax.experimental import pallas as pl
from jax.experimental.pallas import tpu as pltpu
from jax.experimental.pallas import tpu_sc as plsc

VOCAB = 100000
MAXLEN = 60
D = 144
BATCH = 16384
SEQ = 50
NV = D // 16

NC, NS = 2, 16
NW = NC * NS
B_PER_W = BATCH // NW
H = 128
NH = B_PER_W // H
NSTEP = SEQ * NH


def _hsum(v, lanes):
    for s in (8, 4, 2, 1):
        v = v + v.at[lanes ^ s].get(mode="promise_in_bounds")
    return v


def _vrsqrt(x):
    i = lax.bitcast_convert_type(x, jnp.int32)
    y = lax.bitcast_convert_type(jnp.int32(0x5F3759DF) - (i >> 1), jnp.float32)
    for _ in range(2):
        y = y * (1.5 - 0.5 * x * y * y)
    return y


def _tree_sum(vs):
    while len(vs) > 1:
        vs = [vs[i] + vs[i + 1] for i in range(0, len(vs) - 1, 2)] + (
            [vs[-1]] if len(vs) % 2 else [])
    return vs[0]


def _sc_body(xt_hbm, tok_hbm, pos_hbm, gamma_hbm, beta_hbm, out_hbm,
             idx_v, in0_v, in1_v, out0_v, out1_v, pos_v, gb_v,
             gsem0, gsem1, ssem):
    wid = lax.axis_index("s") * NC + lax.axis_index("c")
    bbase = wid * B_PER_W
    in_bufs = (in0_v, in1_v)
    out_bufs = (out0_v, out1_v)
    gsems = (gsem0, gsem1)

    pltpu.sync_copy(xt_hbm.at[:, pl.ds(bbase, B_PER_W)], idx_v)
    pltpu.sync_copy(pos_hbm, pos_v)
    pltpu.sync_copy(gamma_hbm, gb_v.at[0])
    pltpu.sync_copy(beta_hbm, gb_v.at[1])

    g_vecs = [gb_v[0, pl.ds(k * 16, 16)] for k in range(NV)]
    b_vecs = [gb_v[1, pl.ds(k * 16, 16)] for k in range(NV)]
    lanes = lax.iota(jnp.int32, 16)
    lanes_hi = lanes >> 3
    lanes_lo = lanes & 7

    def gather(g, b):
        p = g >> 2
        h = g & 3
        pltpu.async_copy(
            tok_hbm.at[idx_v.at[p, pl.ds(h * H, H)]], in_bufs[b], gsems[b])

    def gather_wait(b):
        pltpu.make_async_copy(
            tok_hbm.at[idx_v.at[0, pl.ds(0, H)]], in_bufs[b], gsems[b]).wait()

    def store_wait():
        pltpu.make_async_copy(
            out_hbm.at[0, :, 0, :, :],
            out0_v.at[:, :, pl.ds(0, H)], ssem).wait()

    gather(0, 0)
    gather(1, 1)

    @pl.loop(0, NSTEP, step=2)
    def step_loop(g0):
        for b in range(2):
            g = g0 + b
            p = g >> 2
            h = g & 3
            gather_wait(b)

            @pl.when(g >= 2)
            def _():
                store_wait()

            in_v = in_bufs[b]
            out_v = out_bufs[b]
            pvs = [pos_v[p, pl.ds(k * 16, 16)] for k in range(NV)]

            @pl.loop(0, H, unroll=2)
            def row_loop(t):
                vs = [in_v[t, pl.ds(k * 16, 16)] + pvs[k] for k in range(NV)]
                mean = _hsum(_tree_sum(vs), lanes) * (1.0 / D)
                q = _hsum(_tree_sum([v * v for v in vs]), lanes) * (1.0 / D)
                r = _vrsqrt(q - mean * mean + 1e-5)
                tv = jnp.broadcast_to(t, (16,))
                for k in range(NV):
                    o = (vs[k] - mean) * (r * g_vecs[k]) + b_vecs[k]
                    plsc.store_scatter(
                        out_v, [lanes_hi + 2 * k, lanes_lo, tv], o)

            @pl.when(g + 2 < NSTEP)
            def _():
                gather(g + 2, b)

            pltpu.async_copy(
                out_v.at[:, :, pl.ds(0, H)],
                out_hbm.at[p, :, (4 * wid) + h, :, :], ssem)

    store_wait()
    store_wait()


@jax.jit
def kernel(x, tok_table, pos_table, gamma, beta):
    x_t = x.T.astype(jnp.int32)
    mesh = plsc.VectorSubcoreMesh(core_axis_name="c", subcore_axis_name="s")
    out_t = pl.kernel(
        _sc_body,
        out_type=jax.ShapeDtypeStruct(
            (SEQ, D // 8, BATCH // 128, 8, 128), jnp.float32),
        mesh=mesh,
        compiler_params=pltpu.CompilerParams(
            use_tc_tiling_on_sc=False, needs_layout_passes=False),
        scratch_types=[
            pltpu.VMEM((SEQ, B_PER_W), jnp.int32),
            pltpu.VMEM((H, D), jnp.float32),
            pltpu.VMEM((H, D), jnp.float32),
            pltpu.VMEM((D // 8, 8, H + 1), jnp.float32),
            pltpu.VMEM((D // 8, 8, H + 1), jnp.float32),
            pltpu.VMEM((MAXLEN, D), jnp.float32),
            pltpu.VMEM((2, D), jnp.float32),
            pltpu.SemaphoreType.DMA,
            pltpu.SemaphoreType.DMA,
            pltpu.SemaphoreType.DMA,
        ],
    )(x_t, tok_table, pos_table, gamma, beta)
    return out_t.transpose(2, 4, 0, 1, 3).reshape(BATCH, SEQ, D)
